# baseline (speedup 1.0000x reference)
import functools

import jax
import jax.numpy as jnp
from jax.experimental import pallas as pl
from jax.experimental.pallas import tpu as pltpu
from jax.experimental.pallas import tpu_sc as plsc

E = 8
TOPK = 2
D = 768
DFF = 3072
T = 2048
CAP = 640
FT = 1536
NF = DFF // FT

NW = 32
TPW = T // NW
TRASH = E * CAP
DISP_ROWS = E * CAP + 8


def _dispatch_sc(hidden, dst1, dst2):
    mesh = plsc.VectorSubcoreMesh(core_axis_name="c", subcore_axis_name="s")

    @functools.partial(
        pl.kernel,
        out_type=jax.ShapeDtypeStruct((DISP_ROWS, D // 2), jnp.int32),
        mesh=mesh,
        scratch_types=[
            pltpu.VMEM((TPW,), jnp.int32),
            pltpu.VMEM((TPW,), jnp.int32),
            pltpu.VMEM((TPW, D // 2), jnp.int32),
            pltpu.SemaphoreType.DMA,
            pltpu.SemaphoreType.DMA,
        ],
    )
    def k(hidden_hbm, dst1_hbm, dst2_hbm, disp_hbm, idx1_v, idx2_v, rows_v,
          sem1, sem2):
        wid = jax.lax.axis_index("s") * 2 + jax.lax.axis_index("c")
        base = wid * TPW
        pltpu.sync_copy(dst1_hbm.at[pl.ds(base, TPW)], idx1_v)
        pltpu.sync_copy(dst2_hbm.at[pl.ds(base, TPW)], idx2_v)
        pltpu.sync_copy(hidden_hbm.at[pl.ds(base, TPW)], rows_v)
        c1 = pltpu.async_copy(rows_v, disp_hbm.at[idx1_v], sem1)
        c2 = pltpu.async_copy(rows_v, disp_hbm.at[idx2_v], sem2)
        c1.wait()
        c2.wait()

    return k(hidden, dst1, dst2)


def _combine_gather_sc(eo, cidx1, cidx2):
    mesh = plsc.VectorSubcoreMesh(core_axis_name="c", subcore_axis_name="s")

    @functools.partial(
        pl.kernel,
        out_type=(jax.ShapeDtypeStruct((T, D), jnp.float32),
                  jax.ShapeDtypeStruct((T, D), jnp.float32)),
        mesh=mesh,
        scratch_types=[
            pltpu.VMEM((TPW,), jnp.int32),
            pltpu.VMEM((TPW,), jnp.int32),
            pltpu.VMEM((TPW, D), jnp.float32),
            pltpu.VMEM((TPW, D), jnp.float32),
            pltpu.SemaphoreType.DMA,
            pltpu.SemaphoreType.DMA,
        ],
    )
    def k(eo_hbm, c1_hbm, c2_hbm, a_hbm, b_hbm, i1_v, i2_v, a_v, b_v, s1, s2):
        wid = jax.lax.axis_index("s") * 2 + jax.lax.axis_index("c")
        base = wid * TPW
        pltpu.sync_copy(c1_hbm.at[pl.ds(base, TPW)], i1_v)
        pltpu.sync_copy(c2_hbm.at[pl.ds(base, TPW)], i2_v)
        g1 = pltpu.async_copy(eo_hbm.at[i1_v], a_v, s1)
        g2 = pltpu.async_copy(eo_hbm.at[i2_v], b_v, s2)
        g1.wait()
        g2.wait()
        pltpu.sync_copy(a_v, a_hbm.at[pl.ds(base, TPW)])
        pltpu.sync_copy(b_v, b_hbm.at[pl.ds(base, TPW)])

    return k(eo, cidx1, cidx2)


TB = T // 128


def _routing_body(lg_ref, dst1_ref, dst2_ref, cidx1_ref, cidx2_ref,
                  w1_ref, w2_ref):
    lt = [lg_ref[e] for e in range(E)]
    neg = jnp.float32(-jnp.inf)

    m1 = lt[0]
    for e in range(1, E):
        m1 = jnp.maximum(m1, lt[e])
    e1 = jnp.full((TB, 128), E, jnp.int32)
    for e in range(E - 1, -1, -1):
        e1 = jnp.where(lt[e] == m1, e, e1)

    lt2 = [jnp.where(e1 == e, neg, lt[e]) for e in range(E)]
    m2 = lt2[0]
    for e in range(1, E):
        m2 = jnp.maximum(m2, lt2[e])
    e2 = jnp.full((TB, 128), E, jnp.int32)
    for e in range(E - 1, -1, -1):
        e2 = jnp.where(lt2[e] == m2, e, e2)

    r = jnp.exp(m2 - m1)
    w1 = 1.0 / (1.0 + r)
    w2 = 1.0 - w1

    cparts = [
        ((e1 == e) | (e2 == e)).astype(jnp.float32) for e in range(E)
    ]
    c = jnp.concatenate(cparts, axis=0)
    n = E * TB
    col = jax.lax.broadcasted_iota(jnp.int32, (128, 128), 1)
    row = jax.lax.broadcasted_iota(jnp.int32, (128, 128), 0)
    u_strict = (row < col).astype(jnp.float32)
    pe = jnp.dot(c, u_strict, preferred_element_type=jnp.float32)
    tot = jnp.sum(c, axis=1, keepdims=True)
    rown = jax.lax.broadcasted_iota(jnp.int32, (n, n), 0)
    coln = jax.lax.broadcasted_iota(jnp.int32, (n, n), 1)
    m_blk = ((rown // TB == coln // TB) & (coln < rown)).astype(jnp.float32)
    offs = jnp.dot(m_blk, tot, preferred_element_type=jnp.float32)
    p = pe + offs

    zi = jnp.zeros((TB, 128), jnp.int32)
    pos1, pos2 = zi, zi
    for e in range(E):
        pblk = p[e * TB:(e + 1) * TB, :].astype(jnp.int32)
        pos1 = jnp.where(e1 == e, pblk, pos1)
        pos2 = jnp.where(e2 == e, pblk, pos2)

    keep1 = pos1 < CAP
    keep2 = pos2 < CAP
    base1 = e1 * CAP
    base2 = e2 * CAP
    dst1_ref[...] = jnp.where(keep1, base1 + pos1, TRASH)
    dst2_ref[...] = jnp.where(keep2, base2 + pos2, TRASH)
    cidx1_ref[...] = jnp.where(keep1, base1 + pos1, base1)
    cidx2_ref[...] = jnp.where(keep2, base2 + pos2, base2)
    w1_ref[...] = jnp.where(keep1, w1, 0.0)
    w2_ref[...] = jnp.where(keep2, w2, 0.0)


def _routing(router_logits):
    lg = router_logits.T.reshape(E, TB, 128)
    oi = jax.ShapeDtypeStruct((TB, 128), jnp.int32)
    of = jax.ShapeDtypeStruct((TB, 128), jnp.float32)
    return pl.pallas_call(
        _routing_body,
        out_shape=(oi, oi, oi, oi, of, of),
    )(lg)


RT = 512


def _weighted_sum_body(a_ref, b_ref, w1_ref, w2_ref, o_ref):
    o_ref[...] = a_ref[...] * w1_ref[...] + b_ref[...] * w2_ref[...]


def _weighted_sum(a, b, w1, w2):
    return pl.pallas_call(
        _weighted_sum_body,
        grid=(T // RT,),
        in_specs=[
            pl.BlockSpec((RT, D), lambda i: (i, 0)),
            pl.BlockSpec((RT, D), lambda i: (i, 0)),
            pl.BlockSpec((RT, 1), lambda i: (i, 0)),
            pl.BlockSpec((RT, 1), lambda i: (i, 0)),
        ],
        out_specs=pl.BlockSpec((RT, D), lambda i: (i, 0)),
        out_shape=jax.ShapeDtypeStruct((T, D), jnp.float32),
    )(a, b, w1, w2)


def _mlp_body(disp_ref, w13g_ref, w13u_ref, w2_ref, out_ref):
    f = pl.program_id(1)
    x = disp_ref[...]
    wg = w13g_ref[0].astype(jnp.bfloat16)
    wu = w13u_ref[0].astype(jnp.bfloat16)
    g = jnp.dot(x, wg, preferred_element_type=jnp.float32)
    u = jnp.dot(x, wu, preferred_element_type=jnp.float32)
    act = (g * jax.nn.sigmoid(g)) * u
    w2b = w2_ref[0].astype(jnp.bfloat16)
    part = jnp.dot(act.astype(jnp.bfloat16), w2b,
                   preferred_element_type=jnp.float32)

    @pl.when(f == 0)
    def _():
        out_ref[...] = part

    @pl.when(f != 0)
    def _():
        out_ref[...] += part


@functools.partial(jax.jit, static_argnames=())
def _expert_mlp(disp, w13, w2):
    return pl.pallas_call(
        _mlp_body,
        grid=(E, NF),
        in_specs=[
            pl.BlockSpec((CAP, D), lambda e, f: (e, 0)),
            pl.BlockSpec((1, D, FT), lambda e, f: (e, 0, f)),
            pl.BlockSpec((1, D, FT), lambda e, f: (e, 0, f + NF)),
            pl.BlockSpec((1, FT, D), lambda e, f: (e, f, 0)),
        ],
        out_specs=pl.BlockSpec((CAP, D), lambda e, f: (e, 0)),
        out_shape=jax.ShapeDtypeStruct((E * CAP, D), jnp.float32),
    )(disp, w13, w13, w2)


def kernel(hidden_states, router_logits, w13, w2):
    dst1, dst2, cidx1, cidx2, w1, w2r = _routing(router_logits)

    hb = hidden_states.astype(jnp.bfloat16)
    hi = jax.lax.bitcast_convert_type(hb.reshape(T, D // 2, 2), jnp.int32)
    disp_i = _dispatch_sc(hi, dst1.reshape(T), dst2.reshape(T))
    disp = jax.lax.bitcast_convert_type(
        disp_i, jnp.bfloat16).reshape(DISP_ROWS, D)

    expert_out = _expert_mlp(disp, w13, w2)

    a, b = _combine_gather_sc(expert_out, cidx1.reshape(T), cidx2.reshape(T))
    return _weighted_sum(a, b, w1.reshape(T, 1), w2r.reshape(T, 1))


# speedup vs baseline: 1.8689x; 1.8689x over previous
import functools

import jax
import jax.numpy as jnp
from jax.experimental import pallas as pl
from jax.experimental.pallas import tpu as pltpu
from jax.experimental.pallas import tpu_sc as plsc

E = 8
TOPK = 2
D = 768
DFF = 3072
T = 2048
CAP = 640
FT = 1536
NF = DFF // FT

NW = 32
TPW = T // NW
TRASH = E * CAP
DISP_ROWS = E * CAP + 8


def _dispatch_sc(hidden, dst1, dst2):
    mesh = plsc.VectorSubcoreMesh(core_axis_name="c", subcore_axis_name="s")

    @functools.partial(
        pl.kernel,
        out_type=jax.ShapeDtypeStruct((DISP_ROWS, D), jnp.float32),
        mesh=mesh,
        scratch_types=[
            pltpu.VMEM((TPW,), jnp.int32),
            pltpu.VMEM((TPW,), jnp.int32),
            pltpu.VMEM((TPW, D), jnp.float32),
            pltpu.SemaphoreType.DMA,
            pltpu.SemaphoreType.DMA,
        ],
    )
    def k(hidden_hbm, dst1_hbm, dst2_hbm, disp_hbm, idx1_v, idx2_v, rows_v,
          sem1, sem2):
        wid = jax.lax.axis_index("s") * 2 + jax.lax.axis_index("c")
        base = wid * TPW
        pltpu.sync_copy(dst1_hbm.at[pl.ds(base, TPW)], idx1_v)
        pltpu.sync_copy(dst2_hbm.at[pl.ds(base, TPW)], idx2_v)
        pltpu.sync_copy(hidden_hbm.at[pl.ds(base, TPW)], rows_v)
        c1 = pltpu.async_copy(rows_v, disp_hbm.at[idx1_v], sem1)
        c2 = pltpu.async_copy(rows_v, disp_hbm.at[idx2_v], sem2)
        c1.wait()
        c2.wait()

    return k(hidden, dst1, dst2)


def _combine_gather_sc(eo, cidx1, cidx2):
    mesh = plsc.VectorSubcoreMesh(core_axis_name="c", subcore_axis_name="s")

    @functools.partial(
        pl.kernel,
        out_type=(jax.ShapeDtypeStruct((T, D), jnp.float32),
                  jax.ShapeDtypeStruct((T, D), jnp.float32)),
        mesh=mesh,
        scratch_types=[
            pltpu.VMEM((TPW,), jnp.int32),
            pltpu.VMEM((TPW,), jnp.int32),
            pltpu.VMEM((TPW, D), jnp.float32),
            pltpu.VMEM((TPW, D), jnp.float32),
            pltpu.SemaphoreType.DMA,
            pltpu.SemaphoreType.DMA,
        ],
    )
    def k(eo_hbm, c1_hbm, c2_hbm, a_hbm, b_hbm, i1_v, i2_v, a_v, b_v, s1, s2):
        wid = jax.lax.axis_index("s") * 2 + jax.lax.axis_index("c")
        base = wid * TPW
        pltpu.sync_copy(c1_hbm.at[pl.ds(base, TPW)], i1_v)
        pltpu.sync_copy(c2_hbm.at[pl.ds(base, TPW)], i2_v)
        g1 = pltpu.async_copy(eo_hbm.at[i1_v], a_v, s1)
        g2 = pltpu.async_copy(eo_hbm.at[i2_v], b_v, s2)
        g1.wait()
        g2.wait()
        pltpu.sync_copy(a_v, a_hbm.at[pl.ds(base, TPW)])
        pltpu.sync_copy(b_v, b_hbm.at[pl.ds(base, TPW)])

    return k(eo, cidx1, cidx2)


TB = T // 128


def _routing_body(lg_ref, dst1_ref, dst2_ref, cidx1_ref, cidx2_ref,
                  w1_ref, w2_ref):
    lt = [lg_ref[e] for e in range(E)]
    neg = jnp.float32(-jnp.inf)

    m1 = lt[0]
    for e in range(1, E):
        m1 = jnp.maximum(m1, lt[e])
    e1 = jnp.full((TB, 128), E, jnp.int32)
    for e in range(E - 1, -1, -1):
        e1 = jnp.where(lt[e] == m1, e, e1)

    lt2 = [jnp.where(e1 == e, neg, lt[e]) for e in range(E)]
    m2 = lt2[0]
    for e in range(1, E):
        m2 = jnp.maximum(m2, lt2[e])
    e2 = jnp.full((TB, 128), E, jnp.int32)
    for e in range(E - 1, -1, -1):
        e2 = jnp.where(lt2[e] == m2, e, e2)

    r = jnp.exp(m2 - m1)
    w1 = 1.0 / (1.0 + r)
    w2 = 1.0 - w1

    cparts = [
        ((e1 == e) | (e2 == e)).astype(jnp.float32) for e in range(E)
    ]
    c = jnp.concatenate(cparts, axis=0)
    n = E * TB
    col = jax.lax.broadcasted_iota(jnp.int32, (128, 128), 1)
    row = jax.lax.broadcasted_iota(jnp.int32, (128, 128), 0)
    u_strict = (row < col).astype(jnp.float32)
    pe = jnp.dot(c, u_strict, preferred_element_type=jnp.float32)
    tot = jnp.sum(c, axis=1, keepdims=True)
    rown = jax.lax.broadcasted_iota(jnp.int32, (n, n), 0)
    coln = jax.lax.broadcasted_iota(jnp.int32, (n, n), 1)
    m_blk = ((rown // TB == coln // TB) & (coln < rown)).astype(jnp.float32)
    offs = jnp.dot(m_blk, tot, preferred_element_type=jnp.float32)
    p = pe + offs

    zi = jnp.zeros((TB, 128), jnp.int32)
    pos1, pos2 = zi, zi
    for e in range(E):
        pblk = p[e * TB:(e + 1) * TB, :].astype(jnp.int32)
        pos1 = jnp.where(e1 == e, pblk, pos1)
        pos2 = jnp.where(e2 == e, pblk, pos2)

    keep1 = pos1 < CAP
    keep2 = pos2 < CAP
    base1 = e1 * CAP
    base2 = e2 * CAP
    dst1_ref[...] = jnp.where(keep1, base1 + pos1, TRASH)
    dst2_ref[...] = jnp.where(keep2, base2 + pos2, TRASH)
    cidx1_ref[...] = jnp.where(keep1, base1 + pos1, base1)
    cidx2_ref[...] = jnp.where(keep2, base2 + pos2, base2)
    w1_ref[...] = jnp.where(keep1, w1, 0.0)
    w2_ref[...] = jnp.where(keep2, w2, 0.0)


def _routing(router_logits):
    lg = router_logits.T.reshape(E, TB, 128)
    oi = jax.ShapeDtypeStruct((TB, 128), jnp.int32)
    of = jax.ShapeDtypeStruct((TB, 128), jnp.float32)
    return pl.pallas_call(
        _routing_body,
        out_shape=(oi, oi, oi, oi, of, of),
    )(lg)


RT = 512


def _weighted_sum_body(a_ref, b_ref, w1_ref, w2_ref, o_ref):
    o_ref[...] = a_ref[...] * w1_ref[...] + b_ref[...] * w2_ref[...]


def _weighted_sum(a, b, w1, w2):
    return pl.pallas_call(
        _weighted_sum_body,
        grid=(T // RT,),
        in_specs=[
            pl.BlockSpec((RT, D), lambda i: (i, 0)),
            pl.BlockSpec((RT, D), lambda i: (i, 0)),
            pl.BlockSpec((RT, 1), lambda i: (i, 0)),
            pl.BlockSpec((RT, 1), lambda i: (i, 0)),
        ],
        out_specs=pl.BlockSpec((RT, D), lambda i: (i, 0)),
        out_shape=jax.ShapeDtypeStruct((T, D), jnp.float32),
    )(a, b, w1, w2)


def _mlp_body(disp_ref, w13g_ref, w13u_ref, w2_ref, out_ref):
    f = pl.program_id(1)
    x = disp_ref[...].astype(jnp.bfloat16)
    wg = w13g_ref[0].astype(jnp.bfloat16)
    wu = w13u_ref[0].astype(jnp.bfloat16)
    g = jnp.dot(x, wg, preferred_element_type=jnp.float32)
    u = jnp.dot(x, wu, preferred_element_type=jnp.float32)
    act = (g * jax.nn.sigmoid(g)) * u
    w2b = w2_ref[0].astype(jnp.bfloat16)
    part = jnp.dot(act.astype(jnp.bfloat16), w2b,
                   preferred_element_type=jnp.float32)

    @pl.when(f == 0)
    def _():
        out_ref[...] = part

    @pl.when(f != 0)
    def _():
        out_ref[...] += part


@functools.partial(jax.jit, static_argnames=())
def _expert_mlp(disp, w13, w2):
    return pl.pallas_call(
        _mlp_body,
        grid=(E, NF),
        in_specs=[
            pl.BlockSpec((CAP, D), lambda e, f: (e, 0)),
            pl.BlockSpec((1, D, FT), lambda e, f: (e, 0, f)),
            pl.BlockSpec((1, D, FT), lambda e, f: (e, 0, f + NF)),
            pl.BlockSpec((1, FT, D), lambda e, f: (e, f, 0)),
        ],
        out_specs=pl.BlockSpec((CAP, D), lambda e, f: (e, 0)),
        out_shape=jax.ShapeDtypeStruct((E * CAP, D), jnp.float32),
    )(disp, w13, w13, w2)


def kernel(hidden_states, router_logits, w13, w2):
    dst1, dst2, cidx1, cidx2, w1, w2r = _routing(router_logits)

    disp = _dispatch_sc(hidden_states, dst1.reshape(T), dst2.reshape(T))

    expert_out = _expert_mlp(disp, w13, w2)

    a, b = _combine_gather_sc(expert_out, cidx1.reshape(T), cidx2.reshape(T))
    return _weighted_sum(a, b, w1.reshape(T, 1), w2r.reshape(T, 1))
